# baseline (speedup 1.0000x reference)
import functools

import jax
import jax.numpy as jnp
from jax import lax
from jax.experimental import pallas as pl
from jax.experimental.pallas import tpu as pltpu
from jax.experimental.pallas import tpu_sc as plsc

N_POINTS = 320000
D = 128
BLK = 128
NBLKS = N_POINTS // BLK
NF = 2512
MAXBPT = 160
NBUF = 4
SEGC = 512
RPT = 40
ACC_ROWS = 16 * RPT
NSEG = 1024


def _sc_body(data_hbm, ids_hbm, firsts_hbm, out_hbm,
             gidx_v, rows_v, idxrow_v, dbuf_v, ones_v, fb_v,
             acc_sh, cacc_sh, gsem, ssem, osem, isem):
    c = lax.axis_index("c")
    s = lax.axis_index("s")

    pltpu.async_copy(firsts_hbm, fb_v, isem)

    one16 = jnp.ones((16,), jnp.float32)
    zero16 = jnp.zeros((16,), jnp.float32)

    def _ones_row(r, carry):
        for cc in range(D // 16):
            ones_v[r, pl.ds(16 * cc, 16)] = one16
        return carry
    lax.fori_loop(0, BLK, _ones_row, 0)

    def _zero_row(r, carry):
        for cc in range(D // 16):
            dbuf_v[0, r, pl.ds(16 * cc, 16)] = zero16
        return carry
    lax.fori_loop(0, RPT, _zero_row, 0)

    base = RPT * s
    pltpu.sync_copy(dbuf_v.at[0].at[pl.ds(0, RPT)],
                    acc_sh.at[pl.ds(base, RPT)])
    pltpu.sync_copy(dbuf_v.at[0].at[pl.ds(0, RPT)],
                    cacc_sh.at[pl.ds(base, RPT)])

    pltpu.make_async_copy(firsts_hbm, fb_v, isem).wait()

    one16i = jnp.ones((16,), jnp.int32)
    zero16i = jnp.zeros((16,), jnp.int32)

    def _scan(i, carry):
        c0, c1 = carry
        v = fb_v[pl.ds(16 * i, 16)]
        c0 = c0 + jnp.where(v < SEGC, one16i, zero16i)
        c1 = c1 + jnp.where(v < NSEG, one16i, zero16i)
        return c0, c1

    c0v, c1v = lax.fori_loop(0, NF // 16, _scan, (zero16i, zero16i))
    C0 = c0v[0]
    C1e = c1v[0]
    for i in range(1, 16):
        C0 = C0 + c0v[i]
        C1e = C1e + c1v[i]

    lo = jnp.where(c == 0, 0, jnp.maximum(C0 - 1, 0))
    hi = jnp.where(c == 0, C0, C1e)
    n = hi - lo
    my_n = jnp.maximum((n - s + 15) // 16, 0)

    lane = jnp.arange(16, dtype=jnp.int32)
    for k in range(MAXBPT // 16):
        v = lo + s + 256 * k + 16 * lane
        gidx_v[pl.ds(16 * k, 16)] = jnp.minimum(v, NBLKS - 1)

    pltpu.async_copy(ids_hbm.at[gidx_v], rows_v, isem)
    pltpu.make_async_copy(ids_hbm.at[gidx_v], rows_v, isem).wait()

    plsc.subcore_barrier()

    segbase = SEGC * c

    def _outer(jo, carry):
        j0 = jo * NBUF
        for t in range(NBUF):
            @pl.when(j0 + t < my_n)
            def _():
                b = lo + s + 16 * (j0 + t)
                pltpu.async_copy(
                    data_hbm.at[pl.ds(b * BLK, BLK)], dbuf_v.at[t], gsem)
        for t in range(NBUF):
            @pl.when(j0 + t < my_n)
            def _():
                j = j0 + t
                b = lo + s + 16 * j
                pltpu.make_async_copy(
                    data_hbm.at[pl.ds(b * BLK, BLK)], dbuf_v.at[t], gsem
                ).wait()
                for cc in range(BLK // 16):
                    v = rows_v[j, pl.ds(16 * cc, 16)] - segbase
                    inrange = jnp.logical_and(v >= 0, v < SEGC)
                    idxrow_v[t, pl.ds(16 * cc, 16)] = jnp.where(
                        inrange, v, SEGC)
                pltpu.async_copy(
                    dbuf_v.at[t], acc_sh.at[idxrow_v.at[t]], ssem, add=True)
                pltpu.async_copy(
                    ones_v, cacc_sh.at[idxrow_v.at[t]], osem, add=True)
        for t in range(NBUF):
            @pl.when(j0 + t < my_n)
            def _():
                pltpu.make_async_copy(
                    dbuf_v.at[t], acc_sh.at[idxrow_v.at[t]], ssem).wait()
                pltpu.make_async_copy(
                    ones_v, cacc_sh.at[idxrow_v.at[t]], osem).wait()
        return carry

    lax.fori_loop(0, -(-MAXBPT // NBUF), _outer, 0)

    plsc.subcore_barrier()

    fin = SEGC // 16
    fbase = fin * s
    pltpu.sync_copy(acc_sh.at[pl.ds(fbase, fin)],
                    dbuf_v.at[1].at[pl.ds(0, fin)])
    pltpu.sync_copy(cacc_sh.at[pl.ds(fbase, fin)],
                    dbuf_v.at[2].at[pl.ds(0, fin)])

    def _div_row(r, carry):
        cnt = jnp.maximum(dbuf_v[2, r, pl.ds(0, 16)], 1.0)
        for cc in range(D // 16):
            dbuf_v[1, r, pl.ds(16 * cc, 16)] = (
                dbuf_v[1, r, pl.ds(16 * cc, 16)] / cnt)
        return carry
    lax.fori_loop(0, fin, _div_row, 0)

    pltpu.sync_copy(dbuf_v.at[1].at[pl.ds(0, fin)],
                    out_hbm.at[pl.ds(segbase + fbase, fin)])


_sc_call = functools.partial(
    pl.kernel,
    out_type=jax.ShapeDtypeStruct((NSEG, D), jnp.float32),
    mesh=plsc.VectorSubcoreMesh(core_axis_name="c", subcore_axis_name="s"),
    scratch_types=(
        pltpu.VMEM((MAXBPT,), jnp.int32),
        pltpu.VMEM((MAXBPT, BLK), jnp.int32),
        pltpu.VMEM((NBUF, BLK), jnp.int32),
        pltpu.VMEM((NBUF, BLK, D), jnp.float32),
        pltpu.VMEM((BLK, D), jnp.float32),
        pltpu.VMEM((NF,), jnp.int32),
        pltpu.VMEM_SHARED((ACC_ROWS, D), jnp.float32),
        pltpu.VMEM_SHARED((ACC_ROWS, D), jnp.float32),
        pltpu.SemaphoreType.DMA,
        pltpu.SemaphoreType.DMA,
        pltpu.SemaphoreType.DMA,
        pltpu.SemaphoreType.DMA,
    ),
)(_sc_body)


def kernel(data, segment_ids):
    ids = segment_ids.astype(jnp.int32)
    ids2d = ids.reshape(NBLKS, BLK)
    firsts = jnp.concatenate(
        [ids2d[:, 0], jnp.full((NF - NBLKS,), jnp.int32(2 ** 30))])
    return _sc_call(data, ids2d, firsts)


# speedup vs baseline: 1.1734x; 1.1734x over previous
import functools

import jax
import jax.numpy as jnp
from jax import lax
from jax.experimental import pallas as pl
from jax.experimental.pallas import tpu as pltpu
from jax.experimental.pallas import tpu_sc as plsc

N_POINTS = 320000
D = 128
BLK = 128
NBLKS = N_POINTS // BLK
NF = 2512
MAXBPT = 160
NBUF = 4
SEGC = 512
RPT = 40
ACC_ROWS = 16 * RPT
NSEG = 1024


def _sc_body(data_hbm, ids_hbm, firsts_hbm, out_hbm,
             gidx_v, rows_v, idxrow_v, dbuf_v, ones_v, fb_v,
             acc_sh, cacc_sh, gsem, ssem, osem, isem):
    c = lax.axis_index("c")
    s = lax.axis_index("s")

    pltpu.async_copy(firsts_hbm, fb_v, isem)

    one16 = jnp.ones((16,), jnp.float32)
    zero16 = jnp.zeros((16,), jnp.float32)

    def _ones_row(r, carry):
        for cc in range(D // 16):
            ones_v[r, pl.ds(16 * cc, 16)] = one16
        return carry
    lax.fori_loop(0, BLK, _ones_row, 0)

    def _zero_row(r, carry):
        for cc in range(D // 16):
            dbuf_v[0, r, pl.ds(16 * cc, 16)] = zero16
        return carry
    lax.fori_loop(0, RPT, _zero_row, 0)

    base = RPT * s
    pltpu.sync_copy(dbuf_v.at[0].at[pl.ds(0, RPT)],
                    acc_sh.at[pl.ds(base, RPT)])
    pltpu.sync_copy(dbuf_v.at[0].at[pl.ds(0, RPT)],
                    cacc_sh.at[pl.ds(base, RPT)])

    pltpu.make_async_copy(firsts_hbm, fb_v, isem).wait()

    one16i = jnp.ones((16,), jnp.int32)
    zero16i = jnp.zeros((16,), jnp.int32)

    def _scan(i, carry):
        c0, c1 = carry
        v = fb_v[pl.ds(16 * i, 16)]
        c0 = c0 + jnp.where(v < SEGC, one16i, zero16i)
        c1 = c1 + jnp.where(v < NSEG, one16i, zero16i)
        return c0, c1

    c0v, c1v = lax.fori_loop(0, NF // 16, _scan, (zero16i, zero16i))
    C0 = c0v[0]
    C1e = c1v[0]
    for i in range(1, 16):
        C0 = C0 + c0v[i]
        C1e = C1e + c1v[i]

    lo = jnp.where(c == 0, 0, jnp.maximum(C0 - 1, 0))
    hi = jnp.where(c == 0, C0, C1e)
    n = hi - lo
    my_n = jnp.maximum((n - s + 15) // 16, 0)

    lane = jnp.arange(16, dtype=jnp.int32)
    for k in range(MAXBPT // 16):
        v = lo + s + 256 * k + 16 * lane
        gidx_v[pl.ds(16 * k, 16)] = jnp.minimum(v, NBLKS - 1)

    for k in range(MAXBPT // 16):
        @pl.when(16 * k < my_n)
        def _():
            pltpu.async_copy(ids_hbm.at[gidx_v.at[pl.ds(16 * k, 16)]],
                             rows_v.at[pl.ds(16 * k, 16)], isem)
    for k in range(MAXBPT // 16):
        @pl.when(16 * k < my_n)
        def _():
            pltpu.make_async_copy(ids_hbm.at[gidx_v.at[pl.ds(16 * k, 16)]],
                                  rows_v.at[pl.ds(16 * k, 16)], isem).wait()

    plsc.subcore_barrier()

    segbase = SEGC * c

    def _outer(jo, carry):
        j0 = jo * NBUF
        for t in range(NBUF):
            @pl.when(j0 + t < my_n)
            def _():
                b = lo + s + 16 * (j0 + t)
                pltpu.async_copy(
                    data_hbm.at[pl.ds(b * BLK, BLK)], dbuf_v.at[t], gsem)
        for t in range(NBUF):
            @pl.when(j0 + t < my_n)
            def _():
                j = j0 + t
                b = lo + s + 16 * j
                pltpu.make_async_copy(
                    data_hbm.at[pl.ds(b * BLK, BLK)], dbuf_v.at[t], gsem
                ).wait()
                for cc in range(BLK // 16):
                    v = rows_v[j, pl.ds(16 * cc, 16)] - segbase
                    inrange = jnp.logical_and(v >= 0, v < SEGC)
                    idxrow_v[t, pl.ds(16 * cc, 16)] = jnp.where(
                        inrange, v, SEGC)
                pltpu.async_copy(
                    dbuf_v.at[t], acc_sh.at[idxrow_v.at[t]], ssem, add=True)
                pltpu.async_copy(
                    ones_v, cacc_sh.at[idxrow_v.at[t]], osem, add=True)
        for t in range(NBUF):
            @pl.when(j0 + t < my_n)
            def _():
                pltpu.make_async_copy(
                    dbuf_v.at[t], acc_sh.at[idxrow_v.at[t]], ssem).wait()
                pltpu.make_async_copy(
                    ones_v, cacc_sh.at[idxrow_v.at[t]], osem).wait()
        return carry

    lax.fori_loop(0, -(-MAXBPT // NBUF), _outer, 0)

    plsc.subcore_barrier()

    fin = SEGC // 16
    fbase = fin * s
    pltpu.sync_copy(acc_sh.at[pl.ds(fbase, fin)],
                    dbuf_v.at[1].at[pl.ds(0, fin)])
    pltpu.sync_copy(cacc_sh.at[pl.ds(fbase, fin)],
                    dbuf_v.at[2].at[pl.ds(0, fin)])

    def _div_row(r, carry):
        cnt = jnp.maximum(dbuf_v[2, r, pl.ds(0, 16)], 1.0)
        for cc in range(D // 16):
            dbuf_v[1, r, pl.ds(16 * cc, 16)] = (
                dbuf_v[1, r, pl.ds(16 * cc, 16)] / cnt)
        return carry
    lax.fori_loop(0, fin, _div_row, 0)

    pltpu.sync_copy(dbuf_v.at[1].at[pl.ds(0, fin)],
                    out_hbm.at[pl.ds(segbase + fbase, fin)])


_sc_call = functools.partial(
    pl.kernel,
    out_type=jax.ShapeDtypeStruct((NSEG, D), jnp.float32),
    mesh=plsc.VectorSubcoreMesh(core_axis_name="c", subcore_axis_name="s"),
    scratch_types=(
        pltpu.VMEM((MAXBPT,), jnp.int32),
        pltpu.VMEM((MAXBPT, BLK), jnp.int32),
        pltpu.VMEM((NBUF, BLK), jnp.int32),
        pltpu.VMEM((NBUF, BLK, D), jnp.float32),
        pltpu.VMEM((BLK, D), jnp.float32),
        pltpu.VMEM((NF,), jnp.int32),
        pltpu.VMEM_SHARED((ACC_ROWS, D), jnp.float32),
        pltpu.VMEM_SHARED((ACC_ROWS, D), jnp.float32),
        pltpu.SemaphoreType.DMA,
        pltpu.SemaphoreType.DMA,
        pltpu.SemaphoreType.DMA,
        pltpu.SemaphoreType.DMA,
    ),
)(_sc_body)


def kernel(data, segment_ids):
    ids = segment_ids.astype(jnp.int32)
    ids2d = ids.reshape(NBLKS, BLK)
    firsts = jnp.concatenate(
        [ids2d[:, 0], jnp.full((NF - NBLKS,), jnp.int32(2 ** 30))])
    return _sc_call(data, ids2d, firsts)


# speedup vs baseline: 1.2273x; 1.0459x over previous
import functools

import jax
import jax.numpy as jnp
from jax import lax
from jax.experimental import pallas as pl
from jax.experimental.pallas import tpu as pltpu
from jax.experimental.pallas import tpu_sc as plsc

N_POINTS = 320000
D = 128
BLK = 128
NBLKS = N_POINTS // BLK
NW = 32
BPW = -(-NBLKS // NW)
BPW_PAD = 80
NBUF = 5
ROWS_PER_TILE = 72
ACC_ROWS = 16 * ROWS_PER_TILE
NSEG = 1024
CW = 128


def _sc_body(data_hbm, ids_hbm, sums_hbm, cnts_hbm,
             gidx_v, rows_v, idxrow_v, dbuf_v, ones_v,
             acc_sh, cacc_sh, gsem, ssem, osem, isem):
    c = lax.axis_index("c")
    s = lax.axis_index("s")
    w = s * 2 + c

    for k in range(BPW_PAD // 16):
        jv = jnp.arange(16, dtype=jnp.int32) + 16 * k
        gidx_v[pl.ds(16 * k, 16)] = jnp.minimum(w + NW * jv, NBLKS - 1)

    pltpu.async_copy(ids_hbm.at[gidx_v], rows_v, isem)

    one16 = jnp.ones((16,), jnp.float32)
    zero16 = jnp.zeros((16,), jnp.float32)

    def _ones_row(r, carry):
        for cc in range(CW // 16):
            ones_v[r, pl.ds(16 * cc, 16)] = one16
        return carry
    lax.fori_loop(0, BLK, _ones_row, 0)

    def _zero_row(r, carry):
        for cc in range(D // 16):
            dbuf_v[0, r, pl.ds(16 * cc, 16)] = zero16
        return carry
    lax.fori_loop(0, ROWS_PER_TILE, _zero_row, 0)

    base = ROWS_PER_TILE * s
    pltpu.sync_copy(dbuf_v.at[0].at[pl.ds(0, ROWS_PER_TILE)],
                    acc_sh.at[pl.ds(base, ROWS_PER_TILE)])
    pltpu.sync_copy(dbuf_v.at[0].at[pl.ds(0, ROWS_PER_TILE)],
                    cacc_sh.at[pl.ds(base, ROWS_PER_TILE)])

    pltpu.make_async_copy(ids_hbm.at[gidx_v], rows_v, isem).wait()

    def _count(j, r):
        b = w + NW * j
        first = rows_v[j, pl.ds(0, 16)][0]
        ok = jnp.logical_and(b < NBLKS, first < NSEG)
        return r + jnp.where(ok, 1, 0).astype(jnp.int32)

    nrel = lax.fori_loop(0, BPW, _count, jnp.int32(0))

    plsc.subcore_barrier()

    def _outer(jo, carry):
        j0 = jo * NBUF
        for t in range(NBUF):
            @pl.when(j0 + t < nrel)
            def _():
                b = w + NW * (j0 + t)
                pltpu.async_copy(
                    data_hbm.at[pl.ds(b * BLK, BLK)], dbuf_v.at[t], gsem)
        for t in range(NBUF):
            @pl.when(j0 + t < nrel)
            def _():
                j = j0 + t
                b = w + NW * j
                pltpu.make_async_copy(
                    data_hbm.at[pl.ds(b * BLK, BLK)], dbuf_v.at[t], gsem
                ).wait()
                for cc in range(BLK // 16):
                    v = rows_v[j, pl.ds(16 * cc, 16)]
                    idxrow_v[t, pl.ds(16 * cc, 16)] = jnp.minimum(v, NSEG)
                pltpu.async_copy(
                    dbuf_v.at[t], acc_sh.at[idxrow_v.at[t]], ssem, add=True)
                pltpu.async_copy(
                    ones_v, cacc_sh.at[idxrow_v.at[t]], osem, add=True)
        for t in range(NBUF):
            @pl.when(j0 + t < nrel)
            def _():
                pltpu.make_async_copy(
                    dbuf_v.at[t], acc_sh.at[idxrow_v.at[t]], ssem).wait()
                pltpu.make_async_copy(
                    ones_v, cacc_sh.at[idxrow_v.at[t]], osem).wait()
        return carry

    lax.fori_loop(0, -(-BPW // NBUF), _outer, 0)

    plsc.subcore_barrier()

    out_base = c * ACC_ROWS + base
    pltpu.async_copy(acc_sh.at[pl.ds(base, ROWS_PER_TILE)],
                     sums_hbm.at[pl.ds(out_base, ROWS_PER_TILE)], gsem)
    pltpu.async_copy(cacc_sh.at[pl.ds(base, ROWS_PER_TILE)],
                     cnts_hbm.at[pl.ds(out_base, ROWS_PER_TILE)], ssem)
    pltpu.make_async_copy(acc_sh.at[pl.ds(base, ROWS_PER_TILE)],
                          sums_hbm.at[pl.ds(out_base, ROWS_PER_TILE)],
                          gsem).wait()
    pltpu.make_async_copy(cacc_sh.at[pl.ds(base, ROWS_PER_TILE)],
                          cnts_hbm.at[pl.ds(out_base, ROWS_PER_TILE)],
                          ssem).wait()


_sc_call = functools.partial(
    pl.kernel,
    out_type=(
        jax.ShapeDtypeStruct((2 * ACC_ROWS, D), jnp.float32),
        jax.ShapeDtypeStruct((2 * ACC_ROWS, CW), jnp.float32),
    ),
    mesh=plsc.VectorSubcoreMesh(core_axis_name="c", subcore_axis_name="s"),
    scratch_types=(
        pltpu.VMEM((BPW_PAD,), jnp.int32),
        pltpu.VMEM((BPW_PAD, BLK), jnp.int32),
        pltpu.VMEM((NBUF, BLK), jnp.int32),
        pltpu.VMEM((NBUF, BLK, D), jnp.float32),
        pltpu.VMEM((BLK, CW), jnp.float32),
        pltpu.VMEM_SHARED((ACC_ROWS, D), jnp.float32),
        pltpu.VMEM_SHARED((ACC_ROWS, CW), jnp.float32),
        pltpu.SemaphoreType.DMA,
        pltpu.SemaphoreType.DMA,
        pltpu.SemaphoreType.DMA,
        pltpu.SemaphoreType.DMA,
    ),
)(_sc_body)


def _finalize_body(sums_ref, cnts_ref, out_ref):
    ssum = sums_ref[0:NSEG, :] + sums_ref[ACC_ROWS:ACC_ROWS + NSEG, :]
    cnt = cnts_ref[0:NSEG, 0:1] + cnts_ref[ACC_ROWS:ACC_ROWS + NSEG, 0:1]
    out_ref[...] = ssum / jnp.maximum(cnt, 1.0)


def kernel(data, segment_ids):
    ids2d = segment_ids.astype(jnp.int32).reshape(NBLKS, BLK)
    sums, cnts = _sc_call(data, ids2d)
    return pl.pallas_call(
        _finalize_body,
        out_shape=jax.ShapeDtypeStruct((NSEG, D), jnp.float32),
    )(sums, cnts)
